# baseline (speedup 1.0000x reference)
import functools

import jax
import jax.numpy as jnp
from jax import lax
from jax.experimental import pallas as pl
from jax.experimental.pallas import tpu as pltpu
from jax.experimental.pallas import tpu_sc as plsc

_L = 16
_NC = 2
_NS = 16


@functools.lru_cache(maxsize=None)
def _build(bs, t_count, nbins, nedges):
    nw = _NC * _NS
    per_w = bs // nw
    groups = per_w // _L
    assert per_w * nw == bs and groups * _L == per_w

    mesh = plsc.VectorSubcoreMesh(core_axis_name="c", subcore_axis_name="s")

    @functools.partial(
        pl.kernel,
        out_type=jax.ShapeDtypeStruct((bs,), jnp.float32),
        mesh=mesh,
        compiler_params=pltpu.CompilerParams(
            needs_layout_passes=False,
            skip_device_barrier=True,
        ),
        scratch_types=[
            pltpu.VMEM((t_count * nbins * nbins,), jnp.float32),
            pltpu.VMEM((t_count * 2 * nedges,), jnp.float32),
            pltpu.VMEM((per_w,), jnp.float32),
            pltpu.VMEM((per_w,), jnp.float32),
            pltpu.VMEM((per_w,), jnp.float32),
            pltpu.SemaphoreType.DMA,
            pltpu.SemaphoreType.DMA,
            pltpu.SemaphoreType.DMA,
            pltpu.SemaphoreType.DMA,
        ],
    )
    def tile_kernel(s0_h, s1_h, w_h, b_h, out_h,
                    w_v, b_v, s0_v, s1_v, o_v,
                    w_sem, b_sem, s0_sem, s1_sem):
        wid = lax.axis_index("s") * _NC + lax.axis_index("c")
        base = wid * per_w
        w_copy = pltpu.async_copy(w_h, w_v, w_sem)
        b_copy = pltpu.async_copy(b_h, b_v, b_sem)
        s0_copy = pltpu.async_copy(s0_h.at[pl.ds(base, per_w)], s0_v, s0_sem)
        s1_copy = pltpu.async_copy(s1_h.at[pl.ds(base, per_w)], s1_v, s1_sem)
        s0_copy.wait()
        s1_copy.wait()
        b_copy.wait()
        w_copy.wait()

        lane = jnp.arange(_L, dtype=jnp.int32)
        inv = jnp.float32(nbins + 1.0 / t_count - 1.0)

        def one_group(g, carry):
            rows = g * _L + lane
            s0 = plsc.load_gather(s0_v, [rows])
            s1 = plsc.load_gather(s1_v, [rows])

            def t0_index(u, s, rowbase):
                cand = jnp.minimum(u.astype(jnp.int32), nbins - 1)
                ebase = cand + rowbase
                elo = plsc.load_gather(b_v, [ebase])
                ehi = plsc.load_gather(b_v, [ebase + 1])
                return (cand + (s >= ehi).astype(jnp.int32)
                        - (s < elo).astype(jnp.int32))

            v0 = t0_index(s0 * inv, s0, 0)
            v1 = t0_index(s1 * inv, s1, nedges)
            fbase = v0 * nbins + v1
            acc = plsc.load_gather(w_v, [fbase])
            for t in range(1, t_count):
                et0 = plsc.load_gather(b_v, [v0 + (t * 2 * nedges + 1)])
                et1 = plsc.load_gather(b_v, [v1 + ((t * 2 + 1) * nedges + 1)])
                d0 = jnp.where(s0 >= et0, nbins, 0)
                d1 = jnp.where(s1 >= et1, t * nbins * nbins + 1,
                               t * nbins * nbins)
                acc = acc + plsc.load_gather(w_v, [fbase + d0 + d1])
            plsc.store_scatter(o_v, [rows], acc)
            return carry

        lax.fori_loop(0, groups, one_group, 0)
        pltpu.sync_copy(o_v, out_h.at[pl.ds(base, per_w)])

    return tile_kernel


def kernel(state, weights, bins):
    bs, _ = state.shape
    t_count, nbins, _ = weights.shape
    nedges = bins.shape[-1]

    s0 = jnp.ravel(state[:, 0])
    s1 = jnp.ravel(state[:, 1])

    fn = _build(bs, t_count, nbins, nedges)
    out = fn(s0, s1, weights.reshape(-1), bins.reshape(-1))
    return out[:, None]


# speedup vs baseline: 1.0705x; 1.0705x over previous
import functools

import jax
import jax.numpy as jnp
from jax import lax
from jax.experimental import pallas as pl
from jax.experimental.pallas import tpu as pltpu
from jax.experimental.pallas import tpu_sc as plsc

_L = 16
_NC = 2
_NS = 16


@functools.lru_cache(maxsize=None)
def _build(bs, t_count, nbins, nedges):
    nw = _NC * _NS
    per_w = bs // nw
    groups = per_w // _L
    assert per_w * nw == bs and groups * _L == per_w

    mesh = plsc.VectorSubcoreMesh(core_axis_name="c", subcore_axis_name="s")
    wsz = t_count * nbins * nbins
    bsz = t_count * 2 * nedges

    @functools.partial(
        pl.kernel,
        out_type=jax.ShapeDtypeStruct((bs,), jnp.float32),
        mesh=mesh,
        compiler_params=pltpu.CompilerParams(needs_layout_passes=False),
        scratch_types=[
            pltpu.VMEM((t_count * nbins * nbins,), jnp.float32),
            pltpu.VMEM((t_count * 2 * nedges,), jnp.float32),
            pltpu.VMEM((per_w,), jnp.float32),
            pltpu.VMEM((per_w,), jnp.float32),
            pltpu.VMEM((per_w,), jnp.float32),
            pltpu.SemaphoreType.DMA,
            pltpu.SemaphoreType.DMA,
            pltpu.SemaphoreType.DMA,
            pltpu.SemaphoreType.DMA,
        ],
    )
    def tile_kernel(big_h, out_h,
                    w_v, b_v, s0_v, s1_v, o_v,
                    w_sem, b_sem, s0_sem, s1_sem):
        wid = lax.axis_index("s") * _NC + lax.axis_index("c")
        base = wid * per_w
        w_copy = pltpu.async_copy(big_h.at[pl.ds(0, wsz)], w_v, w_sem)
        b_copy = pltpu.async_copy(big_h.at[pl.ds(wsz, bsz)], b_v, b_sem)
        s0_copy = pltpu.async_copy(
            big_h.at[pl.ds(wsz + bsz + base, per_w)], s0_v, s0_sem)
        s1_copy = pltpu.async_copy(
            big_h.at[pl.ds(wsz + bsz + bs + base, per_w)], s1_v, s1_sem)
        s0_copy.wait()
        s1_copy.wait()
        b_copy.wait()
        w_copy.wait()

        lane = jnp.arange(_L, dtype=jnp.int32)
        inv = jnp.float32(nbins + 1.0 / t_count - 1.0)

        def one_group(g, carry):
            rows = g * _L + lane
            s0 = plsc.load_gather(s0_v, [rows])
            s1 = plsc.load_gather(s1_v, [rows])

            def t0_index(u, s, rowbase):
                cand = jnp.minimum(u.astype(jnp.int32), nbins - 1)
                ebase = cand + rowbase
                elo = plsc.load_gather(b_v, [ebase])
                ehi = plsc.load_gather(b_v, [ebase + 1])
                return (cand + (s >= ehi).astype(jnp.int32)
                        - (s < elo).astype(jnp.int32))

            v0 = t0_index(s0 * inv, s0, 0)
            v1 = t0_index(s1 * inv, s1, nedges)
            fbase = v0 * nbins + v1
            acc = plsc.load_gather(w_v, [fbase])
            for t in range(1, t_count):
                et0 = plsc.load_gather(b_v, [v0 + (t * 2 * nedges + 1)])
                et1 = plsc.load_gather(b_v, [v1 + ((t * 2 + 1) * nedges + 1)])
                d0 = jnp.where(s0 >= et0, nbins, 0)
                d1 = jnp.where(s1 >= et1, t * nbins * nbins + 1,
                               t * nbins * nbins)
                acc = acc + plsc.load_gather(w_v, [fbase + d0 + d1])
            plsc.store_scatter(o_v, [rows], acc)
            return carry

        lax.fori_loop(0, groups, one_group, 0)
        pltpu.sync_copy(o_v, out_h.at[pl.ds(base, per_w)])

    return tile_kernel


def kernel(state, weights, bins):
    bs, _ = state.shape
    t_count, nbins, _ = weights.shape
    nedges = bins.shape[-1]

    big = jnp.concatenate([
        weights.reshape(-1),
        bins.reshape(-1),
        state[:, 0],
        state[:, 1],
    ])

    fn = _build(bs, t_count, nbins, nedges)
    out = fn(big)
    return out[:, None]


# speedup vs baseline: 1.0740x; 1.0033x over previous
import functools

import jax
import jax.numpy as jnp
from jax import lax
from jax.experimental import pallas as pl
from jax.experimental.pallas import tpu as pltpu
from jax.experimental.pallas import tpu_sc as plsc

_L = 16
_NC = 2
_NS = 16


@functools.lru_cache(maxsize=None)
def _build(bs, t_count, nbins, nedges):
    nw = _NC * _NS
    per_w = bs // nw
    groups = per_w // _L
    assert per_w * nw == bs and groups * _L == per_w

    mesh = plsc.VectorSubcoreMesh(core_axis_name="c", subcore_axis_name="s")
    wsz = t_count * nbins * nbins
    bsz = t_count * 2 * nedges

    @functools.partial(
        pl.kernel,
        out_type=jax.ShapeDtypeStruct((bs,), jnp.float32),
        mesh=mesh,
        compiler_params=pltpu.CompilerParams(needs_layout_passes=False),
        scratch_types=[
            pltpu.VMEM((t_count * nbins * nbins,), jnp.float32),
            pltpu.VMEM((t_count * 2 * nedges,), jnp.float32),
            pltpu.VMEM((per_w,), jnp.float32),
            pltpu.VMEM((per_w,), jnp.float32),
            pltpu.VMEM((per_w,), jnp.float32),
            pltpu.VMEM((per_w,), jnp.int32),
            pltpu.VMEM((per_w,), jnp.int32),
            pltpu.SemaphoreType.DMA,
            pltpu.SemaphoreType.DMA,
            pltpu.SemaphoreType.DMA,
            pltpu.SemaphoreType.DMA,
        ],
    )
    def tile_kernel(big_h, out_h,
                    w_v, b_v, s0_v, s1_v, o_v, v0_v, v1_v,
                    w_sem, b_sem, s0_sem, s1_sem):
        wid = lax.axis_index("s") * _NC + lax.axis_index("c")
        base = wid * per_w
        w_copy = pltpu.async_copy(big_h.at[pl.ds(0, wsz)], w_v, w_sem)
        b_copy = pltpu.async_copy(big_h.at[pl.ds(wsz, bsz)], b_v, b_sem)
        s0_copy = pltpu.async_copy(
            big_h.at[pl.ds(wsz + bsz + base, per_w)], s0_v, s0_sem)
        s1_copy = pltpu.async_copy(
            big_h.at[pl.ds(wsz + bsz + bs + base, per_w)], s1_v, s1_sem)
        s0_copy.wait()
        s1_copy.wait()
        b_copy.wait()

        lane = jnp.arange(_L, dtype=jnp.int32)
        inv = jnp.float32(nbins + 1.0 / t_count - 1.0)

        def index_group(g, carry):
            rows = g * _L + lane
            s0 = plsc.load_gather(s0_v, [rows])
            s1 = plsc.load_gather(s1_v, [rows])

            def t0_index(u, s, rowbase):
                cand = jnp.minimum(u.astype(jnp.int32), nbins - 1)
                ebase = cand + rowbase
                elo = plsc.load_gather(b_v, [ebase])
                ehi = plsc.load_gather(b_v, [ebase + 1])
                return (cand + (s >= ehi).astype(jnp.int32)
                        - (s < elo).astype(jnp.int32))

            plsc.store_scatter(v0_v, [rows], t0_index(s0 * inv, s0, 0))
            plsc.store_scatter(v1_v, [rows], t0_index(s1 * inv, s1, nedges))
            return carry

        lax.fori_loop(0, groups, index_group, 0)
        w_copy.wait()

        def sum_group(g, carry):
            rows = g * _L + lane
            s0 = plsc.load_gather(s0_v, [rows])
            s1 = plsc.load_gather(s1_v, [rows])
            v0 = plsc.load_gather(v0_v, [rows])
            v1 = plsc.load_gather(v1_v, [rows])
            fbase = v0 * nbins + v1
            acc = plsc.load_gather(w_v, [fbase])
            for t in range(1, t_count):
                et0 = plsc.load_gather(b_v, [v0 + (t * 2 * nedges + 1)])
                et1 = plsc.load_gather(b_v, [v1 + ((t * 2 + 1) * nedges + 1)])
                d0 = jnp.where(s0 >= et0, nbins, 0)
                d1 = jnp.where(s1 >= et1, t * nbins * nbins + 1,
                               t * nbins * nbins)
                acc = acc + plsc.load_gather(w_v, [fbase + d0 + d1])
            plsc.store_scatter(o_v, [rows], acc)
            return carry

        lax.fori_loop(0, groups, sum_group, 0)
        pltpu.sync_copy(o_v, out_h.at[pl.ds(base, per_w)])

    return tile_kernel


def kernel(state, weights, bins):
    bs, _ = state.shape
    t_count, nbins, _ = weights.shape
    nedges = bins.shape[-1]

    big = jnp.concatenate([
        weights.reshape(-1),
        bins.reshape(-1),
        state[:, 0],
        state[:, 1],
    ])

    fn = _build(bs, t_count, nbins, nedges)
    out = fn(big)
    return out[:, None]
